# baseline (speedup 1.0000x reference)
import functools

import jax
import jax.numpy as jnp
from jax import lax
from jax.experimental import pallas as pl
from jax.experimental.pallas import tpu as pltpu
from jax.experimental.pallas import tpu_sc as plsc

H, W = 32, 64
HW = H * W
KH, KW = 6, 10
NSQ = 16
C1 = 64
C2 = 64
TD = 128
FCOL = 67
TROWS = HW + 8
SENT = HW
DIST2 = 100.0
QB = 1024


def _make_sc_front():
    mesh = plsc.VectorSubcoreMesh(core_axis_name="c", subcore_axis_name="s")

    @functools.partial(
        pl.kernel,
        out_type=[
            jax.ShapeDtypeStruct((NSQ, HW, TD), jnp.float32),
            jax.ShapeDtypeStruct((H, NSQ, W), jnp.float32),
        ],
        mesh=mesh,
        scratch_types=[
            pltpu.VMEM((4, HW), jnp.float32),
            pltpu.VMEM((4, H + 8, W + 16), jnp.float32),
            pltpu.VMEM((NSQ, W), jnp.int32),
            pltpu.VMEM((1, NSQ, W), jnp.float32),
            pltpu.VMEM((NSQ // 2, W, TD), jnp.float32),
            pltpu.SemaphoreType.DMA,
        ],
    )
    def sc_front(q_hbm, cpad_hbm, table_hbm, gath_hbm, euc_hbm,
                 qbuf, cbuf, idx_buf, euc_v, rows_v, sem):
        h = lax.axis_index("s") * 2 + lax.axis_index("c")
        pltpu.sync_copy(q_hbm, qbuf)
        pltpu.sync_copy(cpad_hbm, cbuf)

        lane = lax.iota(jnp.int32, 16)
        for g in range(4):
            w0 = g * 16
            qx = qbuf[0, pl.ds(h * W + w0, 16)]
            qy = qbuf[1, pl.ds(h * W + w0, 16)]
            qz = qbuf[2, pl.ds(h * W + w0, 16)]
            wvec = w0 + lane
            pq2 = (qx * qx + qy * qy) + qz * qz

            sd = tuple(jnp.full((16,), 3e38, jnp.float32) for _ in range(NSQ))
            si = tuple(jnp.full((16,), SENT, jnp.int32) for _ in range(NSQ))

            for dhi in range(KH):
                row = h - 3 + dhi
                rbase = row * W
                prow = row + 4

                def body(dwi, carry, prow=prow, rbase=rbase,
                         qx=qx, qy=qy, qz=qz, wvec=wvec, w0=w0):
                    sd, si = carry
                    start = w0 + 3 + dwi
                    cx = cbuf[0, prow, pl.ds(start, 16)]
                    cy = cbuf[1, prow, pl.ds(start, 16)]
                    cz = cbuf[2, prow, pl.ds(start, 16)]
                    dx = cx - qx
                    dy = cy - qy
                    dz = cz - qz
                    d2 = (dx * dx + dy * dy) + dz * dz
                    col = (wvec - 5) + dwi
                    nz = ((jnp.abs(cx) + jnp.abs(cy)) + jnp.abs(cz)) > 0.0
                    ok = jnp.logical_and(nz, d2 < DIST2)
                    d = jnp.where(ok, d2, jnp.float32(1e10))
                    i = jnp.where(ok, rbase + col, jnp.int32(SENT))
                    nsd, nsi = [], []
                    for s in range(NSQ):
                        osd, osi = sd[s], si[s]
                        lt = jnp.logical_or(
                            d < osd,
                            jnp.logical_and(d == osd, i < osi))
                        nsd.append(jnp.where(lt, d, osd))
                        nsi.append(jnp.where(lt, i, osi))
                        d = jnp.where(lt, osd, d)
                        i = jnp.where(lt, osi, i)
                    return tuple(nsd), tuple(nsi)

                sd, si = lax.fori_loop(0, KW, body, (sd, si))

            for s in range(NSQ):
                idx_buf[s, pl.ds(w0, 16)] = si[s]
                euc_v[0, s, pl.ds(w0, 16)] = jnp.where(sd[s] < 1e9, sd[s], pq2)

        for s0 in (0, NSQ // 2):
            copies = [
                pltpu.async_copy(table_hbm.at[idx_buf.at[s0 + s]],
                                 rows_v.at[s], sem)
                for s in range(NSQ // 2)
            ]
            for cp in copies:
                cp.wait()
            pltpu.sync_copy(
                rows_v, gath_hbm.at[pl.ds(s0, NSQ // 2), pl.ds(h * W, W), :])
        pltpu.sync_copy(euc_v, euc_hbm.at[pl.ds(h, 1), :, :])

    return sc_front


_SC_FRONT_CACHE = []


def _sc_front(qpad, cpad, table):
    if not _SC_FRONT_CACHE:
        _SC_FRONT_CACHE.append(_make_sc_front())
    return _SC_FRONT_CACHE[0](qpad, cpad, table)


def _tc_body(gath, euc2, x1, p1, wg1, wq1, w91, b1, w2, b2, w3, b3,
             wge, wqe, w9e, be, w4, b4, w5, b5, out):
    g = gath[...]
    m = 1.0 - g[:, :, FCOL:FCOL + 1]
    gm = (g * m).reshape(NSQ * QB, TD)
    euc3 = jnp.sqrt(euc2[...] + 1e-20)[..., None]

    def eterm(w):
        return (euc3 * w[...][None]).reshape(NSQ * QB, w.shape[-1])
    qf = jnp.concatenate([x1[...], p1[...]], axis=-1)

    def dot(x, w):
        return jnp.dot(x, w[...], preferred_element_type=jnp.float32)

    def bdot(q, w):
        r = dot(q, w)
        return jnp.broadcast_to(r[None], (NSQ, QB, r.shape[-1])).reshape(
            NSQ * QB, r.shape[-1])

    l1 = jnp.maximum(
        dot(gm, wg1) + bdot(qf, wq1) + eterm(w91) + b1[...], 0.0)
    l2 = jnp.maximum(dot(l1, w2) + b2[...], 0.0)
    l3 = jnp.maximum(dot(l2, w3) + b3[...], 0.0)
    enc = jnp.maximum(
        dot(gm, wge) + bdot(qf, wqe) + eterm(w9e) + be[...], 0.0)
    pc = jnp.concatenate([enc, l3], axis=-1)
    l4 = jnp.maximum(dot(pc, w4) + b4[...], 0.0)
    l5 = jnp.maximum(dot(l4, w5) + b5[...], 0.0)
    pc3 = l5.reshape(NSQ, QB, 64)
    neg = jnp.where(m == 1.0, pc3, jnp.float32(-1e10))
    mx = jnp.max(neg, axis=0, keepdims=True)
    ex = jnp.exp(neg - mx)
    wq = ex / jnp.sum(ex, axis=0, keepdims=True)
    out[...] = jnp.sum(wq * l3.reshape(NSQ, QB, 64), axis=0)


def _tc_dense(gath, euc2, x1, p1, mats):
    grid = HW // QB
    full = lambda shape: pl.BlockSpec(shape, lambda i: (0,) * len(shape))
    in_specs = [
        pl.BlockSpec((NSQ, QB, TD), lambda i: (0, i, 0)),
        pl.BlockSpec((NSQ, QB), lambda i: (0, i)),
        pl.BlockSpec((QB, 3), lambda i: (i, 0)),
        pl.BlockSpec((QB, C1), lambda i: (i, 0)),
    ] + [full(m.shape) for m in mats]
    return pl.pallas_call(
        _tc_body,
        grid=(grid,),
        in_specs=in_specs,
        out_specs=pl.BlockSpec((QB, 64), lambda i: (i, 0)),
        out_shape=jax.ShapeDtypeStruct((HW, 64), jnp.float32),
    )(gath, euc2, x1, p1, *mats)


def _fold(p):
    w = (p['W'] * p['g'][:, None]).T
    b = (p['b'] * p['g'] + p['beta'])[None, :]
    return w, b


def _layer_split(p):
    wf, b = _fold(p)
    cin, cout = wf.shape
    wg = jnp.zeros((TD, cout), jnp.float32)
    wg = wg.at[0:3].set(wf[3:6] + wf[6:9])
    wq = jnp.zeros((3 + C1, cout), jnp.float32)
    wq = wq.at[0:3].set(wf[0:3] - wf[6:9])
    if cin > 10:
        wg = wg.at[3:3 + C2].set(wf[10 + C1:])
        wq = wq.at[3:].set(wf[10:10 + C1])
    return wg, wq, wf[9], b


def kernel(warped_xyz1_proj, xyz2_proj, points1_proj, points2_proj, params):
    x1 = warped_xyz1_proj.reshape(HW, 3)
    p1 = points1_proj.reshape(HW, C1)
    x2 = xyz2_proj.reshape(HW, 3)
    p2 = points2_proj.reshape(HW, C2)

    q_planes = jnp.pad(x1.T, ((0, 1), (0, 0)))
    c_img = xyz2_proj.reshape(H, W, 3).transpose(2, 0, 1)
    cpad = jnp.pad(c_img, ((0, 1), (4, 4), (8, 8)))
    body = jnp.concatenate(
        [x2, p2, jnp.zeros((HW, TD - 3 - C2), jnp.float32)], axis=-1)
    sent = jnp.zeros((TROWS - HW, TD), jnp.float32).at[0, FCOL].set(1.0)
    table = jnp.concatenate([body, sent], axis=0)

    gath, euc2 = _sc_front(q_planes, cpad, table)
    euc2 = euc2.transpose(1, 0, 2).reshape(NSQ, HW)

    mp = params['mlp1']
    wg1, wq1, w91, b1 = _layer_split(mp[0])
    w2, b2 = _fold(mp[1])
    w3, b3 = _fold(mp[2])
    wge, wqe, w9e, be = _layer_split(params['pi_enc'])
    w4, b4 = _fold(params['mlp2'][0])
    w5, b5 = _fold(params['mlp2'][1])
    mats = [wg1, wq1, w91[None, :], b1, w2, b2, w3, b3,
            wge, wqe, w9e[None, :], be, w4, b4, w5, b5]

    out = _tc_dense(gath, euc2, x1, p1, mats)
    return out.reshape(1, H, W, 64)


# speedup vs baseline: 1.0280x; 1.0280x over previous
import functools

import jax
import jax.numpy as jnp
from jax import lax
from jax.experimental import pallas as pl
from jax.experimental.pallas import tpu as pltpu
from jax.experimental.pallas import tpu_sc as plsc

H, W = 32, 64
HW = H * W
KH, KW = 6, 10
NSQ = 16
C1 = 64
C2 = 64
TD = 128
FCOL = 67
TROWS = HW + 8
SENT = HW
DIST2 = 100.0
QB = 512


def _make_sc_front():
    mesh = plsc.VectorSubcoreMesh(core_axis_name="c", subcore_axis_name="s")

    @functools.partial(
        pl.kernel,
        out_type=[
            jax.ShapeDtypeStruct((NSQ, HW, TD), jnp.float32),
            jax.ShapeDtypeStruct((H, NSQ, W), jnp.float32),
        ],
        mesh=mesh,
        scratch_types=[
            pltpu.VMEM((4, HW), jnp.float32),
            pltpu.VMEM((4, H + 8, W + 16), jnp.float32),
            pltpu.VMEM((NSQ, W), jnp.int32),
            pltpu.VMEM((1, NSQ, W), jnp.float32),
            pltpu.VMEM((NSQ // 2, W, TD), jnp.float32),
            pltpu.SemaphoreType.DMA,
        ],
    )
    def sc_front(q_hbm, cpad_hbm, table_hbm, gath_hbm, euc_hbm,
                 qbuf, cbuf, idx_buf, euc_v, rows_v, sem):
        h = lax.axis_index("s") * 2 + lax.axis_index("c")
        cp_q = pltpu.async_copy(q_hbm, qbuf, sem)
        cp_c = pltpu.async_copy(cpad_hbm, cbuf, sem)
        cp_q.wait()
        cp_c.wait()

        lane = lax.iota(jnp.int32, 16)
        for g in range(4):
            w0 = g * 16
            qx = qbuf[0, pl.ds(h * W + w0, 16)]
            qy = qbuf[1, pl.ds(h * W + w0, 16)]
            qz = qbuf[2, pl.ds(h * W + w0, 16)]
            wvec = w0 + lane
            pq2 = (qx * qx + qy * qy) + qz * qz

            sd = tuple(jnp.full((16,), 3e38, jnp.float32) for _ in range(NSQ))
            si = tuple(jnp.full((16,), SENT, jnp.int32) for _ in range(NSQ))

            for dhi in range(KH):
                row = h - 3 + dhi
                rbase = row * W
                prow = row + 4

                def body(dwi, carry, prow=prow, rbase=rbase,
                         qx=qx, qy=qy, qz=qz, wvec=wvec, w0=w0):
                    sd, si = carry
                    start = w0 + 3 + dwi
                    cx = cbuf[0, prow, pl.ds(start, 16)]
                    cy = cbuf[1, prow, pl.ds(start, 16)]
                    cz = cbuf[2, prow, pl.ds(start, 16)]
                    dx = cx - qx
                    dy = cy - qy
                    dz = cz - qz
                    d2 = (dx * dx + dy * dy) + dz * dz
                    col = (wvec - 5) + dwi
                    nz = ((jnp.abs(cx) + jnp.abs(cy)) + jnp.abs(cz)) > 0.0
                    ok = jnp.logical_and(nz, d2 < DIST2)
                    d = jnp.where(ok, d2, jnp.float32(1e10))
                    i = jnp.where(ok, rbase + col, jnp.int32(SENT))
                    nsd, nsi = [], []
                    for s in range(NSQ):
                        osd, osi = sd[s], si[s]
                        lt = jnp.logical_or(
                            d < osd,
                            jnp.logical_and(d == osd, i < osi))
                        nsd.append(jnp.where(lt, d, osd))
                        nsi.append(jnp.where(lt, i, osi))
                        d = jnp.where(lt, osd, d)
                        i = jnp.where(lt, osi, i)
                    return tuple(nsd), tuple(nsi)

                sd, si = lax.fori_loop(0, KW, body, (sd, si), unroll=2)

            for s in range(NSQ):
                idx_buf[s, pl.ds(w0, 16)] = si[s]
                euc_v[0, s, pl.ds(w0, 16)] = jnp.where(sd[s] < 1e9, sd[s], pq2)

        for s0 in (0, NSQ // 2):
            copies = [
                pltpu.async_copy(table_hbm.at[idx_buf.at[s0 + s]],
                                 rows_v.at[s], sem)
                for s in range(NSQ // 2)
            ]
            for cp in copies:
                cp.wait()
            pltpu.sync_copy(
                rows_v, gath_hbm.at[pl.ds(s0, NSQ // 2), pl.ds(h * W, W), :])
        pltpu.sync_copy(euc_v, euc_hbm.at[pl.ds(h, 1), :, :])

    return sc_front


_SC_FRONT_CACHE = []


def _sc_front(qpad, cpad, table):
    if not _SC_FRONT_CACHE:
        _SC_FRONT_CACHE.append(_make_sc_front())
    return _SC_FRONT_CACHE[0](qpad, cpad, table)


def _tc_body(gath, euc2, x1, p1, wg1, wq1, w91, b1, w2, b2, w3, b3,
             wge, wqe, w9e, be, w4, b4, w5, b5, out):
    g = gath[...]
    m = 1.0 - g[:, :, FCOL:FCOL + 1]
    gm = (g * m).reshape(NSQ * QB, TD)
    euc3 = jnp.sqrt(euc2[...] + 1e-20)[..., None]

    def eterm(w):
        return (euc3 * w[...][None]).reshape(NSQ * QB, w.shape[-1])
    qf = jnp.concatenate([x1[...], p1[...]], axis=-1)

    def dot(x, w):
        return jnp.dot(x, w[...], preferred_element_type=jnp.float32)

    def bdot(q, w):
        r = dot(q, w)
        return jnp.broadcast_to(r[None], (NSQ, QB, r.shape[-1])).reshape(
            NSQ * QB, r.shape[-1])

    l1 = jnp.maximum(
        dot(gm, wg1) + bdot(qf, wq1) + eterm(w91) + b1[...], 0.0)
    l2 = jnp.maximum(dot(l1, w2) + b2[...], 0.0)
    l3 = jnp.maximum(dot(l2, w3) + b3[...], 0.0)
    enc = jnp.maximum(
        dot(gm, wge) + bdot(qf, wqe) + eterm(w9e) + be[...], 0.0)
    pc = jnp.concatenate([enc, l3], axis=-1)
    l4 = jnp.maximum(dot(pc, w4) + b4[...], 0.0)
    l5 = jnp.maximum(dot(l4, w5) + b5[...], 0.0)
    pc3 = l5.reshape(NSQ, QB, 64)
    neg = jnp.where(m == 1.0, pc3, jnp.float32(-1e10))
    mx = jnp.max(neg, axis=0, keepdims=True)
    ex = jnp.exp(neg - mx)
    wq = ex / jnp.sum(ex, axis=0, keepdims=True)
    out[...] = jnp.sum(wq * l3.reshape(NSQ, QB, 64), axis=0)


def _tc_dense(gath, euc2, x1, p1, mats):
    grid = HW // QB
    full = lambda shape: pl.BlockSpec(shape, lambda i: (0,) * len(shape))
    in_specs = [
        pl.BlockSpec((NSQ, QB, TD), lambda i: (0, i, 0)),
        pl.BlockSpec((NSQ, QB), lambda i: (0, i)),
        pl.BlockSpec((QB, 3), lambda i: (i, 0)),
        pl.BlockSpec((QB, C1), lambda i: (i, 0)),
    ] + [full(m.shape) for m in mats]
    return pl.pallas_call(
        _tc_body,
        grid=(grid,),
        in_specs=in_specs,
        out_specs=pl.BlockSpec((QB, 64), lambda i: (i, 0)),
        out_shape=jax.ShapeDtypeStruct((HW, 64), jnp.float32),
    )(gath, euc2, x1, p1, *mats)


def _fold(p):
    w = (p['W'] * p['g'][:, None]).T
    b = (p['b'] * p['g'] + p['beta'])[None, :]
    return w, b


def _layer_split(p):
    wf, b = _fold(p)
    cin, cout = wf.shape
    wg = jnp.zeros((TD, cout), jnp.float32)
    wg = wg.at[0:3].set(wf[3:6] + wf[6:9])
    wq = jnp.zeros((3 + C1, cout), jnp.float32)
    wq = wq.at[0:3].set(wf[0:3] - wf[6:9])
    if cin > 10:
        wg = wg.at[3:3 + C2].set(wf[10 + C1:])
        wq = wq.at[3:].set(wf[10:10 + C1])
    return wg, wq, wf[9], b


def kernel(warped_xyz1_proj, xyz2_proj, points1_proj, points2_proj, params):
    x1 = warped_xyz1_proj.reshape(HW, 3)
    p1 = points1_proj.reshape(HW, C1)
    x2 = xyz2_proj.reshape(HW, 3)
    p2 = points2_proj.reshape(HW, C2)

    q_planes = jnp.pad(x1.T, ((0, 1), (0, 0)))
    c_img = xyz2_proj.reshape(H, W, 3).transpose(2, 0, 1)
    cpad = jnp.pad(c_img, ((0, 1), (4, 4), (8, 8)))
    body = jnp.concatenate(
        [x2, p2, jnp.zeros((HW, TD - 3 - C2), jnp.float32)], axis=-1)
    sent = jnp.zeros((TROWS - HW, TD), jnp.float32).at[0, FCOL].set(1.0)
    table = jnp.concatenate([body, sent], axis=0)

    gath, euc2 = _sc_front(q_planes, cpad, table)
    euc2 = euc2.transpose(1, 0, 2).reshape(NSQ, HW)

    mp = params['mlp1']
    wg1, wq1, w91, b1 = _layer_split(mp[0])
    w2, b2 = _fold(mp[1])
    w3, b3 = _fold(mp[2])
    wge, wqe, w9e, be = _layer_split(params['pi_enc'])
    w4, b4 = _fold(params['mlp2'][0])
    w5, b5 = _fold(params['mlp2'][1])
    mats = [wg1, wq1, w91[None, :], b1, w2, b2, w3, b3,
            wge, wqe, w9e[None, :], be, w4, b4, w5, b5]

    out = _tc_dense(gath, euc2, x1, p1, mats)
    return out.reshape(1, H, W, 64)


# speedup vs baseline: 1.0329x; 1.0048x over previous
import functools

import jax
import jax.numpy as jnp
from jax import lax
from jax.experimental import pallas as pl
from jax.experimental.pallas import tpu as pltpu
from jax.experimental.pallas import tpu_sc as plsc

H, W = 32, 64
HW = H * W
KH, KW = 6, 10
NSQ = 16
C1 = 64
C2 = 64
TD = 128
FCOL = 67
TROWS = HW + 8
SENT = HW
DIST2 = 100.0
QB = 256


def _make_sc_front():
    mesh = plsc.VectorSubcoreMesh(core_axis_name="c", subcore_axis_name="s")

    @functools.partial(
        pl.kernel,
        out_type=[
            jax.ShapeDtypeStruct((NSQ, HW, TD), jnp.float32),
            jax.ShapeDtypeStruct((H, NSQ, W), jnp.float32),
        ],
        mesh=mesh,
        scratch_types=[
            pltpu.VMEM((4, HW), jnp.float32),
            pltpu.VMEM((4, H + 8, W + 16), jnp.float32),
            pltpu.VMEM((NSQ, W), jnp.int32),
            pltpu.VMEM((1, NSQ, W), jnp.float32),
            pltpu.VMEM((NSQ // 2, W, TD), jnp.float32),
            pltpu.SemaphoreType.DMA,
        ],
    )
    def sc_front(q_hbm, cpad_hbm, table_hbm, gath_hbm, euc_hbm,
                 qbuf, cbuf, idx_buf, euc_v, rows_v, sem):
        h = lax.axis_index("s") * 2 + lax.axis_index("c")
        cp_q = pltpu.async_copy(q_hbm, qbuf, sem)
        cp_c = pltpu.async_copy(cpad_hbm, cbuf, sem)
        cp_q.wait()
        cp_c.wait()

        lane = lax.iota(jnp.int32, 16)
        for g in range(4):
            w0 = g * 16
            qx = qbuf[0, pl.ds(h * W + w0, 16)]
            qy = qbuf[1, pl.ds(h * W + w0, 16)]
            qz = qbuf[2, pl.ds(h * W + w0, 16)]
            wvec = w0 + lane
            pq2 = (qx * qx + qy * qy) + qz * qz

            sd = tuple(jnp.full((16,), 3e38, jnp.float32) for _ in range(NSQ))
            si = tuple(jnp.full((16,), SENT, jnp.int32) for _ in range(NSQ))

            for dhi in range(KH):
                row = h - 3 + dhi
                rbase = row * W
                prow = row + 4

                def body(dwi, carry, prow=prow, rbase=rbase,
                         qx=qx, qy=qy, qz=qz, wvec=wvec, w0=w0):
                    sd, si = carry
                    start = w0 + 3 + dwi
                    cx = cbuf[0, prow, pl.ds(start, 16)]
                    cy = cbuf[1, prow, pl.ds(start, 16)]
                    cz = cbuf[2, prow, pl.ds(start, 16)]
                    dx = cx - qx
                    dy = cy - qy
                    dz = cz - qz
                    d2 = (dx * dx + dy * dy) + dz * dz
                    col = (wvec - 5) + dwi
                    nz = ((jnp.abs(cx) + jnp.abs(cy)) + jnp.abs(cz)) > 0.0
                    ok = jnp.logical_and(nz, d2 < DIST2)
                    d = jnp.where(ok, d2, jnp.float32(1e10))
                    i = jnp.where(ok, rbase + col, jnp.int32(SENT))
                    nsd, nsi = [], []
                    for s in range(NSQ):
                        osd, osi = sd[s], si[s]
                        lt = jnp.logical_or(
                            d < osd,
                            jnp.logical_and(d == osd, i < osi))
                        nsd.append(jnp.where(lt, d, osd))
                        nsi.append(jnp.where(lt, i, osi))
                        d = jnp.where(lt, osd, d)
                        i = jnp.where(lt, osi, i)
                    return tuple(nsd), tuple(nsi)

                sd, si = lax.fori_loop(0, KW, body, (sd, si), unroll=2)

            for s in range(NSQ):
                idx_buf[s, pl.ds(w0, 16)] = si[s]
                euc_v[0, s, pl.ds(w0, 16)] = jnp.where(sd[s] < 1e9, sd[s], pq2)

        for s0 in (0, NSQ // 2):
            copies = [
                pltpu.async_copy(table_hbm.at[idx_buf.at[s0 + s]],
                                 rows_v.at[s], sem)
                for s in range(NSQ // 2)
            ]
            for cp in copies:
                cp.wait()
            pltpu.sync_copy(
                rows_v, gath_hbm.at[pl.ds(s0, NSQ // 2), pl.ds(h * W, W), :])
        pltpu.sync_copy(euc_v, euc_hbm.at[pl.ds(h, 1), :, :])

    return sc_front


_SC_FRONT_CACHE = []


def _sc_front(qpad, cpad, table):
    if not _SC_FRONT_CACHE:
        _SC_FRONT_CACHE.append(_make_sc_front())
    return _SC_FRONT_CACHE[0](qpad, cpad, table)


def _tc_body(gath, euc2, x1, p1, wg1, wq1, w91, b1, w2, b2, w3, b3,
             wge, wqe, w9e, be, w4, b4, w5, b5, out):
    g = gath[...]
    m = 1.0 - g[:, :, FCOL:FCOL + 1]
    gm = (g * m).reshape(NSQ * QB, TD)
    euc3 = jnp.sqrt(euc2[...] + 1e-20)[..., None]

    def eterm(w):
        return (euc3 * w[...][None]).reshape(NSQ * QB, w.shape[-1])
    qf = jnp.concatenate([x1[...], p1[...]], axis=-1)

    def dot(x, w):
        return jnp.dot(x, w[...], preferred_element_type=jnp.float32)

    def bdot(q, w):
        r = dot(q, w)
        return jnp.broadcast_to(r[None], (NSQ, QB, r.shape[-1])).reshape(
            NSQ * QB, r.shape[-1])

    l1 = jnp.maximum(
        dot(gm, wg1) + bdot(qf, wq1) + eterm(w91) + b1[...], 0.0)
    l2 = jnp.maximum(dot(l1, w2) + b2[...], 0.0)
    l3 = jnp.maximum(dot(l2, w3) + b3[...], 0.0)
    enc = jnp.maximum(
        dot(gm, wge) + bdot(qf, wqe) + eterm(w9e) + be[...], 0.0)
    pc = jnp.concatenate([enc, l3], axis=-1)
    l4 = jnp.maximum(dot(pc, w4) + b4[...], 0.0)
    l5 = jnp.maximum(dot(l4, w5) + b5[...], 0.0)
    pc3 = l5.reshape(NSQ, QB, 64)
    neg = jnp.where(m == 1.0, pc3, jnp.float32(-1e10))
    mx = jnp.max(neg, axis=0, keepdims=True)
    ex = jnp.exp(neg - mx)
    wq = ex / jnp.sum(ex, axis=0, keepdims=True)
    out[...] = jnp.sum(wq * l3.reshape(NSQ, QB, 64), axis=0)


def _tc_dense(gath, euc2, x1, p1, mats):
    grid = HW // QB
    full = lambda shape: pl.BlockSpec(shape, lambda i: (0,) * len(shape))
    in_specs = [
        pl.BlockSpec((NSQ, QB, TD), lambda i: (0, i, 0)),
        pl.BlockSpec((NSQ, QB), lambda i: (0, i)),
        pl.BlockSpec((QB, 3), lambda i: (i, 0)),
        pl.BlockSpec((QB, C1), lambda i: (i, 0)),
    ] + [full(m.shape) for m in mats]
    return pl.pallas_call(
        _tc_body,
        grid=(grid,),
        in_specs=in_specs,
        out_specs=pl.BlockSpec((QB, 64), lambda i: (i, 0)),
        out_shape=jax.ShapeDtypeStruct((HW, 64), jnp.float32),
    )(gath, euc2, x1, p1, *mats)


def _fold(p):
    w = (p['W'] * p['g'][:, None]).T
    b = (p['b'] * p['g'] + p['beta'])[None, :]
    return w, b


def _layer_split(p):
    wf, b = _fold(p)
    cin, cout = wf.shape
    wg = jnp.zeros((TD, cout), jnp.float32)
    wg = wg.at[0:3].set(wf[3:6] + wf[6:9])
    wq = jnp.zeros((3 + C1, cout), jnp.float32)
    wq = wq.at[0:3].set(wf[0:3] - wf[6:9])
    if cin > 10:
        wg = wg.at[3:3 + C2].set(wf[10 + C1:])
        wq = wq.at[3:].set(wf[10:10 + C1])
    return wg, wq, wf[9], b


def kernel(warped_xyz1_proj, xyz2_proj, points1_proj, points2_proj, params):
    x1 = warped_xyz1_proj.reshape(HW, 3)
    p1 = points1_proj.reshape(HW, C1)
    x2 = xyz2_proj.reshape(HW, 3)
    p2 = points2_proj.reshape(HW, C2)

    q_planes = jnp.pad(x1.T, ((0, 1), (0, 0)))
    c_img = xyz2_proj.reshape(H, W, 3).transpose(2, 0, 1)
    cpad = jnp.pad(c_img, ((0, 1), (4, 4), (8, 8)))
    body = jnp.concatenate(
        [x2, p2, jnp.zeros((HW, TD - 3 - C2), jnp.float32)], axis=-1)
    sent = jnp.zeros((TROWS - HW, TD), jnp.float32).at[0, FCOL].set(1.0)
    table = jnp.concatenate([body, sent], axis=0)

    gath, euc2 = _sc_front(q_planes, cpad, table)
    euc2 = euc2.transpose(1, 0, 2).reshape(NSQ, HW)

    mp = params['mlp1']
    wg1, wq1, w91, b1 = _layer_split(mp[0])
    w2, b2 = _fold(mp[1])
    w3, b3 = _fold(mp[2])
    wge, wqe, w9e, be = _layer_split(params['pi_enc'])
    w4, b4 = _fold(params['mlp2'][0])
    w5, b5 = _fold(params['mlp2'][1])
    mats = [wg1, wq1, w91[None, :], b1, w2, b2, w3, b3,
            wge, wqe, w9e[None, :], be, w4, b4, w5, b5]

    out = _tc_dense(gath, euc2, x1, p1, mats)
    return out.reshape(1, H, W, 64)


# speedup vs baseline: 1.0390x; 1.0059x over previous
import functools

import jax
import jax.numpy as jnp
from jax import lax
from jax.experimental import pallas as pl
from jax.experimental.pallas import tpu as pltpu
from jax.experimental.pallas import tpu_sc as plsc

H, W = 32, 64
HW = H * W
KH, KW = 6, 10
NSQ = 16
C1 = 64
C2 = 64
TD = 128
FCOL = 67
TROWS = HW + 8
SENT = HW
DIST2 = 100.0
QB = 256


def _make_sc_front():
    mesh = plsc.VectorSubcoreMesh(core_axis_name="c", subcore_axis_name="s")

    @functools.partial(
        pl.kernel,
        out_type=[
            jax.ShapeDtypeStruct((NSQ, HW, TD), jnp.float32),
            jax.ShapeDtypeStruct((H, NSQ, W), jnp.float32),
        ],
        mesh=mesh,
        scratch_types=[
            pltpu.VMEM((4, HW), jnp.float32),
            pltpu.VMEM((4, H + 8, W + 16), jnp.float32),
            pltpu.VMEM((NSQ, W), jnp.int32),
            pltpu.VMEM((1, NSQ, W), jnp.float32),
            pltpu.VMEM((2, NSQ // 4, W, TD), jnp.float32),
            pltpu.SemaphoreType.DMA,
            pltpu.SemaphoreType.DMA,
        ],
    )
    def sc_front(q_hbm, cpad_hbm, table_hbm, gath_hbm, euc_hbm,
                 qbuf, cbuf, idx_buf, euc_v, rows_v, sem, sem_o):
        h = lax.axis_index("s") * 2 + lax.axis_index("c")
        cp_q = pltpu.async_copy(q_hbm, qbuf, sem)
        cp_c = pltpu.async_copy(cpad_hbm, cbuf, sem)
        cp_q.wait()
        cp_c.wait()

        lane = lax.iota(jnp.int32, 16)
        for g in range(4):
            w0 = g * 16
            qx = qbuf[0, pl.ds(h * W + w0, 16)]
            qy = qbuf[1, pl.ds(h * W + w0, 16)]
            qz = qbuf[2, pl.ds(h * W + w0, 16)]
            wvec = w0 + lane
            pq2 = (qx * qx + qy * qy) + qz * qz

            sd = tuple(jnp.full((16,), 3e38, jnp.float32) for _ in range(NSQ))
            si = tuple(jnp.full((16,), SENT, jnp.int32) for _ in range(NSQ))

            for dhi in range(KH):
                row = h - 3 + dhi
                rbase = row * W
                prow = row + 4

                def body(dwi, carry, prow=prow, rbase=rbase,
                         qx=qx, qy=qy, qz=qz, wvec=wvec, w0=w0):
                    sd, si = carry
                    start = w0 + 3 + dwi
                    cx = cbuf[0, prow, pl.ds(start, 16)]
                    cy = cbuf[1, prow, pl.ds(start, 16)]
                    cz = cbuf[2, prow, pl.ds(start, 16)]
                    dx = cx - qx
                    dy = cy - qy
                    dz = cz - qz
                    d2 = (dx * dx + dy * dy) + dz * dz
                    col = (wvec - 5) + dwi
                    nz = ((jnp.abs(cx) + jnp.abs(cy)) + jnp.abs(cz)) > 0.0
                    ok = jnp.logical_and(nz, d2 < DIST2)
                    d = jnp.where(ok, d2, jnp.float32(1e10))
                    i = jnp.where(ok, rbase + col, jnp.int32(SENT))
                    nsd, nsi = [], []
                    for s in range(NSQ):
                        osd, osi = sd[s], si[s]
                        lt = jnp.logical_or(
                            d < osd,
                            jnp.logical_and(d == osd, i < osi))
                        nsd.append(jnp.where(lt, d, osd))
                        nsi.append(jnp.where(lt, i, osi))
                        d = jnp.where(lt, osd, d)
                        i = jnp.where(lt, osi, i)
                    return tuple(nsd), tuple(nsi)

                sd, si = lax.fori_loop(0, KW, body, (sd, si), unroll=2)

            for s in range(NSQ):
                idx_buf[s, pl.ds(w0, 16)] = si[s]
                euc_v[0, s, pl.ds(w0, 16)] = jnp.where(sd[s] < 1e9, sd[s], pq2)

        NW4 = NSQ // 4

        def fire_gathers(wv):
            return [
                pltpu.async_copy(table_hbm.at[idx_buf.at[wv * NW4 + s]],
                                 rows_v.at[wv % 2, s], sem)
                for s in range(NW4)
            ]
        gcp = {0: fire_gathers(0), 1: fire_gathers(1)}
        ocp = {}
        for wv in range(4):
            for cp in gcp[wv]:
                cp.wait()
            ocp[wv] = pltpu.async_copy(
                rows_v.at[wv % 2],
                gath_hbm.at[pl.ds(wv * NW4, NW4), pl.ds(h * W, W), :], sem_o)
            if wv + 2 < 4:
                ocp[wv].wait()
                gcp[wv + 2] = fire_gathers(wv + 2)
        ocp[2].wait()
        ocp[3].wait()
        pltpu.sync_copy(euc_v, euc_hbm.at[pl.ds(h, 1), :, :])

    return sc_front


_SC_FRONT_CACHE = []


def _sc_front(qpad, cpad, table):
    if not _SC_FRONT_CACHE:
        _SC_FRONT_CACHE.append(_make_sc_front())
    return _SC_FRONT_CACHE[0](qpad, cpad, table)


def _tc_body(gath, euc2, x1, p1, wg1, wq1, w91, b1, w2, b2, w3, b3,
             wge, wqe, w9e, be, w4, b4, w5, b5, out):
    g = gath[...]
    m = 1.0 - g[:, :, FCOL:FCOL + 1]
    gm = (g * m).reshape(NSQ * QB, TD)
    euc3 = jnp.sqrt(euc2[...] + 1e-20)[..., None]

    def eterm(w):
        return (euc3 * w[...][None]).reshape(NSQ * QB, w.shape[-1])
    qf = jnp.concatenate([x1[...], p1[...]], axis=-1)

    def dot(x, w):
        return jnp.dot(x, w[...], preferred_element_type=jnp.float32)

    def bdot(q, w):
        r = dot(q, w)
        return jnp.broadcast_to(r[None], (NSQ, QB, r.shape[-1])).reshape(
            NSQ * QB, r.shape[-1])

    l1 = jnp.maximum(
        dot(gm, wg1) + bdot(qf, wq1) + eterm(w91) + b1[...], 0.0)
    l2 = jnp.maximum(dot(l1, w2) + b2[...], 0.0)
    l3 = jnp.maximum(dot(l2, w3) + b3[...], 0.0)
    enc = jnp.maximum(
        dot(gm, wge) + bdot(qf, wqe) + eterm(w9e) + be[...], 0.0)
    pc = jnp.concatenate([enc, l3], axis=-1)
    l4 = jnp.maximum(dot(pc, w4) + b4[...], 0.0)
    l5 = jnp.maximum(dot(l4, w5) + b5[...], 0.0)
    pc3 = l5.reshape(NSQ, QB, 64)
    neg = jnp.where(m == 1.0, pc3, jnp.float32(-1e10))
    mx = jnp.max(neg, axis=0, keepdims=True)
    ex = jnp.exp(neg - mx)
    wq = ex / jnp.sum(ex, axis=0, keepdims=True)
    out[...] = jnp.sum(wq * l3.reshape(NSQ, QB, 64), axis=0)


def _tc_dense(gath, euc2, x1, p1, mats):
    grid = HW // QB
    full = lambda shape: pl.BlockSpec(shape, lambda i: (0,) * len(shape))
    in_specs = [
        pl.BlockSpec((NSQ, QB, TD), lambda i: (0, i, 0)),
        pl.BlockSpec((NSQ, QB), lambda i: (0, i)),
        pl.BlockSpec((QB, 3), lambda i: (i, 0)),
        pl.BlockSpec((QB, C1), lambda i: (i, 0)),
    ] + [full(m.shape) for m in mats]
    return pl.pallas_call(
        _tc_body,
        grid=(grid,),
        in_specs=in_specs,
        out_specs=pl.BlockSpec((QB, 64), lambda i: (i, 0)),
        out_shape=jax.ShapeDtypeStruct((HW, 64), jnp.float32),
    )(gath, euc2, x1, p1, *mats)


def _fold(p):
    w = (p['W'] * p['g'][:, None]).T
    b = (p['b'] * p['g'] + p['beta'])[None, :]
    return w, b


def _layer_split(p):
    wf, b = _fold(p)
    cin, cout = wf.shape
    wg = jnp.zeros((TD, cout), jnp.float32)
    wg = wg.at[0:3].set(wf[3:6] + wf[6:9])
    wq = jnp.zeros((3 + C1, cout), jnp.float32)
    wq = wq.at[0:3].set(wf[0:3] - wf[6:9])
    if cin > 10:
        wg = wg.at[3:3 + C2].set(wf[10 + C1:])
        wq = wq.at[3:].set(wf[10:10 + C1])
    return wg, wq, wf[9], b


def kernel(warped_xyz1_proj, xyz2_proj, points1_proj, points2_proj, params):
    x1 = warped_xyz1_proj.reshape(HW, 3)
    p1 = points1_proj.reshape(HW, C1)
    x2 = xyz2_proj.reshape(HW, 3)
    p2 = points2_proj.reshape(HW, C2)

    q_planes = jnp.pad(x1.T, ((0, 1), (0, 0)))
    c_img = xyz2_proj.reshape(H, W, 3).transpose(2, 0, 1)
    cpad = jnp.pad(c_img, ((0, 1), (4, 4), (8, 8)))
    body = jnp.concatenate(
        [x2, p2, jnp.zeros((HW, TD - 3 - C2), jnp.float32)], axis=-1)
    sent = jnp.zeros((TROWS - HW, TD), jnp.float32).at[0, FCOL].set(1.0)
    table = jnp.concatenate([body, sent], axis=0)

    gath, euc2 = _sc_front(q_planes, cpad, table)
    euc2 = euc2.transpose(1, 0, 2).reshape(NSQ, HW)

    mp = params['mlp1']
    wg1, wq1, w91, b1 = _layer_split(mp[0])
    w2, b2 = _fold(mp[1])
    w3, b3 = _fold(mp[2])
    wge, wqe, w9e, be = _layer_split(params['pi_enc'])
    w4, b4 = _fold(params['mlp2'][0])
    w5, b5 = _fold(params['mlp2'][1])
    mats = [wg1, wq1, w91[None, :], b1, w2, b2, w3, b3,
            wge, wqe, w9e[None, :], be, w4, b4, w5, b5]

    out = _tc_dense(gath, euc2, x1, p1, mats)
    return out.reshape(1, H, W, 64)
